# baseline (speedup 1.0000x reference)
import functools

import jax
import jax.numpy as jnp
from jax import lax
from jax.experimental import pallas as pl
from jax.experimental.pallas import tpu as pltpu
from jax.experimental.pallas import tpu_sc as plsc

_N = 10000
_E = 320000
_D = 128
_NC = 2
_NS = 16
_NW = _NC * _NS
_K = 128
_NP = 10240
_RPT = _NP // _NS
_EP = 327680
_CPW = _EP // _NW // _K
_ERows = _EP // _K

_mesh = plsc.VectorSubcoreMesh(core_axis_name="c", subcore_axis_name="s")


def _zero_rows(rows_ref, nrows):
    z = jnp.zeros((16,), jnp.float32)

    def body(i, _):
        r = i // 8
        col = (i % 8) * 16
        rows_ref[r, pl.ds(col, 16)] = z
        return _

    lax.fori_loop(0, nrows * 8, body, None)


def _copy_tile_slice(src_at, dst_at, stage_ref, rbase):
    for k in range(_RPT // _K):
        pltpu.sync_copy(src_at(pl.ds(rbase + k * _K, _K)), stage_ref)
        pltpu.sync_copy(stage_ref, dst_at(pl.ds(rbase + k * _K, _K)))


_HB = 16384
_HPT = _HB // _NS


def _deg_body(pidx_hbm, out_hbm, idx_v, hist, tmp, acc):
    c = lax.axis_index("c")
    s = lax.axis_index("s")
    wid = c * _NS + s
    ones16 = jnp.ones((16,), jnp.float32)
    z16 = jnp.zeros((16,), jnp.float32)

    def zero(i, _):
        hist[pl.ds(i * 16, 16)] = z16
        return _

    lax.fori_loop(0, _HB // 16, zero, None)

    pltpu.sync_copy(pidx_hbm.at[pl.ds(wid * _CPW, _CPW)], idx_v)

    def chunk(cc, _):
        row = idx_v.at[cc]
        for g in range(_K // 16):
            d16 = lax.shift_right_logical(row[pl.ds(g * 16, 16)], 14)
            plsc.addupdate_scatter(hist, [d16], ones16)
        return _

    lax.fori_loop(0, _CPW, chunk, None)

    pltpu.sync_copy(hist, acc.at[pl.ds(s * _HB, _HB)])
    plsc.subcore_barrier()
    rb = s * _HPT
    pltpu.sync_copy(acc.at[pl.ds(rb, _HPT)], hist.at[pl.ds(0, _HPT)])

    def hsum(h, _):
        pltpu.sync_copy(acc.at[pl.ds(h * _HB + rb, _HPT)], tmp)
        for j in range(_HPT // 16):
            sl = pl.ds(j * 16, 16)
            hist[sl] = hist[sl] + tmp[sl]
        return _

    lax.fori_loop(1, _NS, hsum, None)
    obase = pl.multiple_of(c * _HB + rb, 8)
    pltpu.sync_copy(hist.at[pl.ds(0, _HPT)], out_hbm.at[pl.ds(obase, _HPT)])


_deg_kernel = pl.kernel(
    _deg_body,
    out_type=jax.ShapeDtypeStruct((_NC * _HB,), jnp.float32),
    mesh=_mesh,
    compiler_params=pltpu.CompilerParams(needs_layout_passes=False),
    scratch_types=[
        pltpu.VMEM((_CPW, _K), jnp.int32),
        pltpu.VMEM((_HB,), jnp.float32),
        pltpu.VMEM((_HPT,), jnp.float32),
        pltpu.VMEM_SHARED((_NS * _HB,), jnp.float32),
    ],
)


def _agg_body(src_hbm, dst_hbm, hs_hbm, out_hbm,
              sidx, didx, rows0, sem0, acc):
    c = lax.axis_index("c")
    s = lax.axis_index("s")
    wid = c * _NS + s
    rbase = s * _RPT

    _zero_rows(rows0, _K)
    for k in range(_RPT // _K):
        pltpu.sync_copy(rows0, acc.at[pl.ds(rbase + k * _K, _K)])
    plsc.subcore_barrier()

    ebase = wid * _CPW * _K

    def chunk(cc, _):
        base = pl.multiple_of(ebase + cc * _K, 8)
        pltpu.sync_copy(src_hbm.at[pl.ds(base, _K)], sidx)
        pltpu.sync_copy(dst_hbm.at[pl.ds(base, _K)], didx)
        pltpu.async_copy(hs_hbm.at[sidx], rows0, sem0).wait()
        pltpu.sync_copy(rows0, acc.at[didx], add=True)
        return _

    lax.fori_loop(0, _CPW, chunk, None)
    plsc.subcore_barrier()

    @pl.when(c == 0)
    def _():
        _copy_tile_slice(lambda d: acc.at[d], lambda d: out_hbm.at[0, d], rows0, rbase)

    @pl.when(c == 1)
    def _():
        _copy_tile_slice(lambda d: acc.at[d], lambda d: out_hbm.at[1, d], rows0, rbase)


_agg_kernel = pl.kernel(
    _agg_body,
    out_type=jax.ShapeDtypeStruct((_NC, _NP, _D), jnp.float32),
    mesh=_mesh,
    scratch_types=[
        pltpu.VMEM((_K,), jnp.int32),
        pltpu.VMEM((_K,), jnp.int32),
        pltpu.VMEM((_K, _D), jnp.float32),
        pltpu.SemaphoreType.DMA,
        pltpu.VMEM_SHARED((_NP, _D), jnp.float32),
    ],
)


def _mm_body(x_ref, w_ref, o_ref):
    o_ref[...] = jnp.dot(x_ref[...], w_ref[...], preferred_element_type=jnp.float32)


_mm_call = pl.pallas_call(
    _mm_body, out_shape=jax.ShapeDtypeStruct((_N, _D), jnp.float32))


def _scale_body(d0_ref, d1_ref, h_ref, hs_ref, dinv_ref):
    d = d0_ref[...] + d1_ref[...] + 1.0
    dinv = lax.rsqrt(d)
    dinv_ref[...] = dinv
    hs_ref[0:_N, :] = h_ref[...] * dinv
    hs_ref[_N:_NP, :] = jnp.zeros((_NP - _N, _D), jnp.float32)


_scale_call = pl.pallas_call(
    _scale_body,
    out_shape=[
        jax.ShapeDtypeStruct((_NP, _D), jnp.float32),
        jax.ShapeDtypeStruct((_N, 1), jnp.float32),
    ],
)


def _bn_prelu(g, gamma, beta, a):
    mu = jnp.mean(g, axis=0, keepdims=True)
    va = jnp.mean((g - mu) ** 2, axis=0, keepdims=True)
    y = (g - mu) * lax.rsqrt(va + 1e-5) * gamma + beta
    return jnp.where(y >= 0.0, y, a * y)


def _mid_body(msg_ref, hs_ref, dinv_ref, b_ref, gamma_ref, beta_ref, a_ref,
              w_ref, o_ref):
    agg = msg_ref[0, 0:_N, :] + msg_ref[1, 0:_N, :] + hs_ref[0:_N, :]
    g = agg * dinv_ref[...] + b_ref[...]
    t = _bn_prelu(g, gamma_ref[...], beta_ref[...], a_ref[...])
    h2 = jnp.dot(t, w_ref[...], preferred_element_type=jnp.float32)
    o_ref[0:_N, :] = h2 * dinv_ref[...]
    o_ref[_N:_NP, :] = jnp.zeros((_NP - _N, _D), jnp.float32)


_mid_call = pl.pallas_call(
    _mid_body, out_shape=jax.ShapeDtypeStruct((_NP, _D), jnp.float32))


def _fin_body(msg_ref, hs_ref, dinv_ref, b_ref, gamma_ref, beta_ref, a_ref,
              o_ref):
    agg = msg_ref[0, 0:_N, :] + msg_ref[1, 0:_N, :] + hs_ref[0:_N, :]
    g = agg * dinv_ref[...] + b_ref[...]
    o_ref[...] = _bn_prelu(g, gamma_ref[...], beta_ref[...], a_ref[...])


_fin_call = pl.pallas_call(
    _fin_body, out_shape=jax.ShapeDtypeStruct((_N, _D), jnp.float32))


def kernel(x, edge_index, W1, b1, gamma1, beta1, a1, W2, b2, gamma2, beta2, a2):
    pad = jnp.full((2, _EP - _E), _N, jnp.int32)
    eip = jnp.concatenate([edge_index, pad], axis=1)
    src1d = eip[0]
    dst1d = eip[1]
    pidx2d = (eip[0] | (eip[1] << 14)).reshape(_ERows, _K)
    b1r = b1.reshape(1, _D)
    b2r = b2.reshape(1, _D)
    g1r = gamma1.reshape(1, _D)
    g2r = gamma2.reshape(1, _D)
    be1r = beta1.reshape(1, _D)
    be2r = beta2.reshape(1, _D)
    a1r = a1.reshape(1, 1)
    a2r = a2.reshape(1, 1)

    degacc = _deg_kernel(pidx2d)
    d0 = degacc[:_N].reshape(-1, 1)
    d1 = degacc[_HB:_HB + _N].reshape(-1, 1)
    h1 = _mm_call(x, W1)
    hs1, dinv = _scale_call(d0, d1, h1)
    msg1 = _agg_kernel(src1d, dst1d, hs1)
    hs2 = _mid_call(msg1, hs1, dinv, b1r, g1r, be1r, a1r, W2)
    msg2 = _agg_kernel(src1d, dst1d, hs2)
    return _fin_call(msg2, hs2, dinv, b2r, g2r, be2r, a2r)


# speedup vs baseline: 4.0577x; 4.0577x over previous
import functools

import jax
import jax.numpy as jnp
from jax import lax
from jax.experimental import pallas as pl
from jax.experimental.pallas import tpu as pltpu
from jax.experimental.pallas import tpu_sc as plsc

_N = 10000
_E = 320000
_D = 128
_NC = 2
_NS = 16
_NW = _NC * _NS
_K = 128
_NP = 10240
_RPT = _NP // _NS
_EP = 327680
_CPW = _EP // _NW // _K
_ERows = _EP // _K

_mesh = plsc.VectorSubcoreMesh(core_axis_name="c", subcore_axis_name="s")


def _zero_rows(rows_ref, nrows):
    z = jnp.zeros((16,), jnp.float32)

    def body(i, _):
        r = i // 8
        col = (i % 8) * 16
        rows_ref[r, pl.ds(col, 16)] = z
        return _

    lax.fori_loop(0, nrows * 8, body, None)


def _copy_tile_slice(src_at, dst_at, stage_ref, rbase):
    for k in range(_RPT // _K):
        pltpu.sync_copy(src_at(pl.ds(rbase + k * _K, _K)), stage_ref)
        pltpu.sync_copy(stage_ref, dst_at(pl.ds(rbase + k * _K, _K)))


_HB = 16384
_HPT = _HB // _NS


def _deg_body(pidx_hbm, out_hbm, idx_v, hist, tmp, acc):
    c = lax.axis_index("c")
    s = lax.axis_index("s")
    wid = c * _NS + s
    ones16 = jnp.ones((16,), jnp.float32)
    z16 = jnp.zeros((16,), jnp.float32)

    def zero(i, _):
        hist[pl.ds(i * 16, 16)] = z16
        return _

    lax.fori_loop(0, _HB // 16, zero, None)

    pltpu.sync_copy(pidx_hbm.at[pl.ds(wid * _CPW, _CPW)], idx_v)

    def chunk(cc, _):
        row = idx_v.at[cc]
        for g in range(_K // 16):
            d16 = lax.shift_right_logical(row[pl.ds(g * 16, 16)], 14)
            plsc.addupdate_scatter(hist, [d16], ones16)
        return _

    lax.fori_loop(0, _CPW, chunk, None)

    pltpu.sync_copy(hist, acc.at[pl.ds(s * _HB, _HB)])
    plsc.subcore_barrier()
    rb = s * _HPT
    pltpu.sync_copy(acc.at[pl.ds(rb, _HPT)], hist.at[pl.ds(0, _HPT)])

    def hsum(h, _):
        pltpu.sync_copy(acc.at[pl.ds(h * _HB + rb, _HPT)], tmp)
        for j in range(_HPT // 16):
            sl = pl.ds(j * 16, 16)
            hist[sl] = hist[sl] + tmp[sl]
        return _

    lax.fori_loop(1, _NS, hsum, None)
    obase = pl.multiple_of(c * _HB + rb, 8)
    pltpu.sync_copy(hist.at[pl.ds(0, _HPT)], out_hbm.at[pl.ds(obase, _HPT)])


_deg_kernel = pl.kernel(
    _deg_body,
    out_type=jax.ShapeDtypeStruct((_NC * _HB,), jnp.float32),
    mesh=_mesh,
    compiler_params=pltpu.CompilerParams(needs_layout_passes=False),
    scratch_types=[
        pltpu.VMEM((_CPW, _K), jnp.int32),
        pltpu.VMEM((_HB,), jnp.float32),
        pltpu.VMEM((_HPT,), jnp.float32),
        pltpu.VMEM_SHARED((_NS * _HB,), jnp.float32),
    ],
)


def _unpack_chunk(pidx, cc, sref, dref):
    row = pidx.at[cc]
    for g in range(_K // 16):
        sl = pl.ds(g * 16, 16)
        p = row[sl]
        sref[sl] = lax.bitwise_and(p, 16383)
        dref[sl] = lax.shift_right_logical(p, 14)


def _agg_body(pidx_hbm, hs_hbm, out_hbm,
              pidx, sidx0, didx0, sidx1, didx1, rows0, rows1, sem0, sem1, acc):
    c = lax.axis_index("c")
    s = lax.axis_index("s")
    wid = c * _NS + s
    rbase = s * _RPT

    _zero_rows(rows0, _K)
    for k in range(_RPT // _K):
        pltpu.sync_copy(rows0, acc.at[pl.ds(rbase + k * _K, _K)])
    plsc.subcore_barrier()

    pltpu.sync_copy(pidx_hbm.at[pl.ds(wid * _CPW, _CPW)], pidx)

    _unpack_chunk(pidx, 0, sidx0, didx0)
    pltpu.async_copy(hs_hbm.at[sidx0], rows0, sem0)

    def chunk2(g, _):
        c0 = g * 2
        _unpack_chunk(pidx, c0 + 1, sidx1, didx1)
        pltpu.async_copy(hs_hbm.at[sidx1], rows1, sem1)
        pltpu.make_async_copy(hs_hbm.at[sidx0], rows0, sem0).wait()
        pltpu.sync_copy(rows0, acc.at[didx0], add=True)

        @pl.when(c0 + 2 < _CPW)
        def _():
            _unpack_chunk(pidx, c0 + 2, sidx0, didx0)
            pltpu.async_copy(hs_hbm.at[sidx0], rows0, sem0)

        pltpu.make_async_copy(hs_hbm.at[sidx1], rows1, sem1).wait()
        pltpu.sync_copy(rows1, acc.at[didx1], add=True)
        return _

    lax.fori_loop(0, _CPW // 2, chunk2, None)
    plsc.subcore_barrier()

    @pl.when(c == 0)
    def _():
        _copy_tile_slice(lambda d: acc.at[d], lambda d: out_hbm.at[0, d], rows0, rbase)

    @pl.when(c == 1)
    def _():
        _copy_tile_slice(lambda d: acc.at[d], lambda d: out_hbm.at[1, d], rows0, rbase)


_agg_kernel = pl.kernel(
    _agg_body,
    out_type=jax.ShapeDtypeStruct((_NC, _NP, _D), jnp.float32),
    mesh=_mesh,
    scratch_types=[
        pltpu.VMEM((_CPW, _K), jnp.int32),
        pltpu.VMEM((_K,), jnp.int32),
        pltpu.VMEM((_K,), jnp.int32),
        pltpu.VMEM((_K,), jnp.int32),
        pltpu.VMEM((_K,), jnp.int32),
        pltpu.VMEM((_K, _D), jnp.float32),
        pltpu.VMEM((_K, _D), jnp.float32),
        pltpu.SemaphoreType.DMA,
        pltpu.SemaphoreType.DMA,
        pltpu.VMEM_SHARED((_NP, _D), jnp.float32),
    ],
)


def _mm_body(x_ref, w_ref, o_ref):
    o_ref[...] = jnp.dot(x_ref[...], w_ref[...], preferred_element_type=jnp.float32)


_mm_call = pl.pallas_call(
    _mm_body, out_shape=jax.ShapeDtypeStruct((_N, _D), jnp.float32))


def _scale_body(d0_ref, d1_ref, h_ref, hs_ref, dinv_ref):
    d = d0_ref[...] + d1_ref[...] + 1.0
    dinv = lax.rsqrt(d)
    dinv_ref[...] = dinv
    hs_ref[0:_N, :] = h_ref[...] * dinv
    hs_ref[_N:_NP, :] = jnp.zeros((_NP - _N, _D), jnp.float32)


_scale_call = pl.pallas_call(
    _scale_body,
    out_shape=[
        jax.ShapeDtypeStruct((_NP, _D), jnp.float32),
        jax.ShapeDtypeStruct((_N, 1), jnp.float32),
    ],
)


def _bn_prelu(g, gamma, beta, a):
    mu = jnp.mean(g, axis=0, keepdims=True)
    va = jnp.mean((g - mu) ** 2, axis=0, keepdims=True)
    y = (g - mu) * lax.rsqrt(va + 1e-5) * gamma + beta
    return jnp.where(y >= 0.0, y, a * y)


def _mid_body(msg_ref, hs_ref, dinv_ref, b_ref, gamma_ref, beta_ref, a_ref,
              w_ref, o_ref):
    agg = msg_ref[0, 0:_N, :] + msg_ref[1, 0:_N, :] + hs_ref[0:_N, :]
    g = agg * dinv_ref[...] + b_ref[...]
    t = _bn_prelu(g, gamma_ref[...], beta_ref[...], a_ref[...])
    h2 = jnp.dot(t, w_ref[...], preferred_element_type=jnp.float32)
    o_ref[0:_N, :] = h2 * dinv_ref[...]
    o_ref[_N:_NP, :] = jnp.zeros((_NP - _N, _D), jnp.float32)


_mid_call = pl.pallas_call(
    _mid_body, out_shape=jax.ShapeDtypeStruct((_NP, _D), jnp.float32))


def _fin_body(msg_ref, hs_ref, dinv_ref, b_ref, gamma_ref, beta_ref, a_ref,
              o_ref):
    agg = msg_ref[0, 0:_N, :] + msg_ref[1, 0:_N, :] + hs_ref[0:_N, :]
    g = agg * dinv_ref[...] + b_ref[...]
    o_ref[...] = _bn_prelu(g, gamma_ref[...], beta_ref[...], a_ref[...])


_fin_call = pl.pallas_call(
    _fin_body, out_shape=jax.ShapeDtypeStruct((_N, _D), jnp.float32))


def kernel(x, edge_index, W1, b1, gamma1, beta1, a1, W2, b2, gamma2, beta2, a2):
    padv = _N + jax.lax.rem(jnp.arange(_EP - _E, dtype=jnp.int32),
                            jnp.int32(_NP - _N))
    eip = jnp.concatenate([edge_index, jnp.stack([padv, padv])], axis=1)
    pidx2d = (eip[0] | (eip[1] << 14)).reshape(_ERows, _K)
    b1r = b1.reshape(1, _D)
    b2r = b2.reshape(1, _D)
    g1r = gamma1.reshape(1, _D)
    g2r = gamma2.reshape(1, _D)
    be1r = beta1.reshape(1, _D)
    be2r = beta2.reshape(1, _D)
    a1r = a1.reshape(1, 1)
    a2r = a2.reshape(1, 1)

    degacc = _deg_kernel(pidx2d)
    d0 = degacc[:_N].reshape(-1, 1)
    d1 = degacc[_HB:_HB + _N].reshape(-1, 1)
    h1 = _mm_call(x, W1)
    hs1, dinv = _scale_call(d0, d1, h1)
    msg1 = _agg_kernel(pidx2d, hs1)
    hs2 = _mid_call(msg1, hs1, dinv, b1r, g1r, be1r, a1r, W2)
    msg2 = _agg_kernel(pidx2d, hs2)
    return _fin_call(msg2, hs2, dinv, b2r, g2r, be2r, a2r)
